# baseline (speedup 1.0000x reference)
import functools

import jax
import jax.numpy as jnp
from jax import lax
from jax.experimental import pallas as pl
from jax.experimental.pallas import tpu as pltpu, tpu_sc as plsc

_TOP_K_PERCENT = 0.25
_HEM_STEP = 100000
_STEP_CONST = 50000

_B = 16
_N = 512 * 512
_ROWS2D = 2048
_LANES = 128

_HALF = _N // 2
_K = int((min(1.0, _STEP_CONST / _HEM_STEP) * _TOP_K_PERCENT
          + (1.0 - min(1.0, _STEP_CONST / _HEM_STEP))) * _N)
_HALFW = _HALF // 2
_CH = 8192
_NCH = _HALFW // _CH
_NB12 = 4096
_NB3 = 16
_U = 8


def _count_loss_body(x_ref, g_ref, out_ref, cnt_ref):
    i = pl.program_id(0)

    @pl.when(i == 0)
    def _init():
        cnt_ref[0] = 0.0

    @pl.when(i < _B)
    def _count():
        cnt_ref[0] += jnp.sum(g_ref[0, :, :].astype(jnp.float32))

    @pl.when(i >= _B)
    def _loss():
        pos = cnt_ref[0]
        total = float(_B * _N)
        wp = (total - pos) / total
        wn = pos / total
        x0 = x_ref[0, 0, :, :]
        x1 = x_ref[0, 1, :, :]
        g = g_ref[0, :, :]
        d = x0 - x1
        z = jnp.where(g == 1, d, -d)
        sp = jnp.maximum(z, 0.0) + jnp.log1p(jnp.exp(-jnp.abs(z)))
        w = jnp.where(g == 1, wp, wn)
        out_ref[0, :, :] = (w * sp).astype(jnp.bfloat16)


def _zero(ref, nbins, dtype):
    def body(j, _):
        ref[pl.ds(j * 16, 16)] = jnp.zeros((16,), dtype)
        return 0
    lax.fori_loop(0, nbins // 16, body, 0)


def _scan_hist(ref, nbins, k_rank):
    nv = nbins // 16

    def body(j, carry):
        acc, found, b, sb1 = carry
        jj = nv - 1 - j
        v = ref[pl.ds(jj * 16, 16)]
        svec = lax.rev(jnp.cumsum(lax.rev(v, (0,))), (0,)) + acc
        ge = svec >= k_rank
        trues = jnp.sum(ge.astype(jnp.int32))
        found_now = jnp.logical_and(jnp.logical_not(found), trues > 0)
        sb1_here = jnp.maximum(jnp.max(jnp.where(ge, jnp.int32(-1), svec)), acc)
        b = jnp.where(found_now, jj * 16 + trues - 1, b)
        sb1 = jnp.where(found_now, sb1_here, sb1)
        found = jnp.logical_or(found, trues > 0)
        return acc + jnp.sum(v), found, b, sb1

    _, _, b, sb1 = lax.fori_loop(
        0, nv, body, (jnp.int32(0), False, jnp.int32(0), jnp.int32(0))
    )
    return b, sb1


def _sc_select_body(loss_hbm, out_hbm, buf_v, hist_v, part_v,
                    s3_v, stg_v, res_v, sh_i, sh_f, dma_sem):
    cid = lax.axis_index("c")
    sid = lax.axis_index("s")
    row = cid * 8 + sid // 2
    half = sid % 2
    base = row * (_N // 2) + half * _HALFW
    partner = sid + 1 - 2 * half

    def data_pass(pass_body):
        def start(c):
            return pltpu.async_copy(
                loss_hbm.at[pl.ds(base + c * _CH, _CH)],
                buf_v.at[pl.ds((c % 2) * _CH, _CH)],
                dma_sem,
            )

        carry = jnp.float32(0.0)
        desc = start(0)
        for c in range(_NCH):
            desc.wait()
            if c + 1 < _NCH:
                desc = start(c + 1)
            cbase = (c % 2) * _CH

            def vec(off, carry2):
                x = buf_v[pl.ds(cbase + off, 16)]
                carry2 = pass_body(x & 0xFFFF, carry2)
                return pass_body((x >> 16) & 0xFFFF, carry2)
            carry = plsc.parallel_loop(0, _CH, 16, unroll=_U, carry=carry)(vec)
        return carry

    ones = jnp.ones((16,), jnp.int32)

    def merge_hist(nbins):
        pltpu.sync_copy(hist_v.at[pl.ds(0, nbins)], sh_i.at[sid, pl.ds(0, nbins)])
        plsc.subcore_barrier()
        pltpu.sync_copy(sh_i.at[partner, pl.ds(0, nbins)], part_v.at[pl.ds(0, nbins)])

        def madd(j, _):
            d = pl.ds(j * 16, 16)
            hist_v[d] = hist_v[d] + part_v[d]
            return 0
        lax.fori_loop(0, nbins // 16, madd, 0)
        plsc.subcore_barrier()

    _zero(hist_v, _NB12, jnp.int32)

    def p1(key, carry):
        plsc.addupdate_scatter(hist_v, [key >> 3], ones)
        return carry
    data_pass(p1)
    merge_hist(_NB12)
    b1, c1 = _scan_hist(hist_v, _NB12, jnp.int32(_K))
    k2 = _K - c1

    _zero(hist_v, _NB3, jnp.int32)
    _zero(s3_v, _NB3, jnp.float32)

    def p2(key, sum_hi):
        hi12 = key >> 3
        m_eq = hi12 == b1
        val = plsc.bitcast(key << 16, jnp.float32)
        plsc.addupdate_scatter(hist_v, [key & 7], ones, mask=m_eq)
        plsc.addupdate_scatter(s3_v, [key & 7], val, mask=m_eq)
        return sum_hi + jnp.sum(jnp.where(hi12 > b1, val, 0.0))
    sum_hi = data_pass(p2)

    pltpu.sync_copy(hist_v.at[pl.ds(0, _NB3)], sh_i.at[sid, pl.ds(0, _NB3)])
    stg_v[pl.ds(0, 16)] = s3_v[pl.ds(0, 16)]
    stg_v[pl.ds(16, 16)] = jnp.full((16,), sum_hi, jnp.float32)
    pltpu.sync_copy(stg_v.at[pl.ds(0, 32)], sh_f.at[sid, pl.ds(0, 32)])
    plsc.subcore_barrier()
    pltpu.sync_copy(sh_i.at[partner, pl.ds(0, _NB3)], part_v.at[pl.ds(0, _NB3)])
    pltpu.sync_copy(sh_f.at[partner, pl.ds(0, 32)], stg_v.at[pl.ds(0, 32)])

    hist_v[pl.ds(0, 16)] = hist_v[pl.ds(0, 16)] + part_v[pl.ds(0, 16)]
    s3_v[pl.ds(0, 16)] = s3_v[pl.ds(0, 16)] + stg_v[pl.ds(0, 16)]
    sum_hi = sum_hi + jnp.max(stg_v[pl.ds(16, 16)])

    b2, c2 = _scan_hist(hist_v, _NB3, k2)
    t_bits = (b1 << 3) | b2
    t_val = plsc.bitcast(jnp.full((16,), t_bits << 16, jnp.int32), jnp.float32)

    gidx = lax.iota(jnp.int32, 16)
    sum_in = jnp.sum(jnp.where(gidx > b2, s3_v[pl.ds(0, 16)], 0.0))

    row_sum = sum_hi + sum_in + t_val[0] * (k2 - c2).astype(jnp.float32)

    @pl.when(half == 0)
    def _():
        res_v[...] = jnp.full((16,), row_sum, jnp.float32)
        pltpu.sync_copy(res_v, out_hbm.at[row])


_sc_select = functools.partial(
    pl.kernel,
    out_type=jax.ShapeDtypeStruct((_B, 16), jnp.float32),
    mesh=plsc.VectorSubcoreMesh(core_axis_name="c", subcore_axis_name="s"),
    scratch_types=[
        pltpu.VMEM((2 * _CH,), jnp.int32),
        pltpu.VMEM((_NB12,), jnp.int32),
        pltpu.VMEM((_NB12,), jnp.int32),
        pltpu.VMEM((_NB3,), jnp.float32),
        pltpu.VMEM((256,), jnp.float32),
        pltpu.VMEM((16,), jnp.float32),
        pltpu.VMEM_SHARED((16, _NB12), jnp.int32),
        pltpu.VMEM_SHARED((16, 256), jnp.float32),
        pltpu.SemaphoreType.DMA,
    ],
    compiler_params=pltpu.CompilerParams(needs_layout_passes=False),
)(_sc_select_body)


def kernel(pred_logits, gts, step):
    B, C, H, W = pred_logits.shape

    g3 = gts.reshape(B, _ROWS2D, _LANES)
    x4 = pred_logits.reshape(B, C, _ROWS2D, _LANES)

    losses = pl.pallas_call(
        _count_loss_body,
        grid=(2 * B,),
        in_specs=[
            pl.BlockSpec(
                (1, C, _ROWS2D, _LANES),
                lambda i: (jnp.where(i < _B, 0, i - _B), 0, 0, 0),
            ),
            pl.BlockSpec(
                (1, _ROWS2D, _LANES),
                lambda i: (jnp.where(i < _B, i, i - _B), 0, 0),
            ),
        ],
        out_specs=pl.BlockSpec(
            (1, _ROWS2D, _LANES),
            lambda i: (jnp.where(i < _B, 0, i - _B), 0, 0),
        ),
        out_shape=jax.ShapeDtypeStruct((B, _ROWS2D, _LANES), jnp.bfloat16),
        scratch_shapes=[pltpu.SMEM((1,), jnp.float32)],
    )(x4, g3)

    words = lax.bitcast_convert_type(
        losses.reshape(B * _N // 2, 2), jnp.int32
    )
    row_sums = _sc_select(words)[:, 0]

    ratio_t = jnp.minimum(1.0, jnp.asarray(step, jnp.float32) / _HEM_STEP)
    return jnp.sum(row_sums) / float(B * _K) + 0.0 * ratio_t


# speedup vs baseline: 10.3959x; 10.3959x over previous
import functools

import jax
import jax.numpy as jnp
from jax import lax
from jax.experimental import pallas as pl
from jax.experimental.pallas import tpu as pltpu, tpu_sc as plsc

_TOP_K_PERCENT = 0.25
_HEM_STEP = 100000
_STEP_CONST = 50000

_B = 16
_N = 512 * 512
_ROWS2D = 2048
_LANES = 128

_HALF = _N // 2
_K = int((min(1.0, _STEP_CONST / _HEM_STEP) * _TOP_K_PERCENT
          + (1.0 - min(1.0, _STEP_CONST / _HEM_STEP))) * _N)
_HALFW = _HALF // 2
_CH = 8192
_NCH = _HALFW // _CH
_NB12 = 4096
_NB3 = 16
_U = 8


def _count_loss_body(x_ref, g_ref, out_ref, cnt_ref):
    i = pl.program_id(0)

    @pl.when(i == 0)
    def _init():
        cnt_ref[0] = 0.0

    @pl.when(i < _B)
    def _count():
        cnt_ref[0] += jnp.sum(g_ref[0, :, :].astype(jnp.float32))

    @pl.when(i >= _B)
    def _loss():
        pos = cnt_ref[0]
        total = float(_B * _N)
        wp = (total - pos) / total
        wn = pos / total
        x0 = x_ref[0, 0, :, :]
        x1 = x_ref[0, 1, :, :]
        g = g_ref[0, :, :]
        d = x0 - x1
        z = jnp.where(g == 1, d, -d)
        sp = jnp.maximum(z, 0.0) + jnp.log1p(jnp.exp(-jnp.abs(z)))
        w = jnp.where(g == 1, wp, wn)
        loss = (w * sp).astype(jnp.bfloat16)
        l3 = loss.reshape(_ROWS2D // 2, 2, _LANES)
        a = jax.lax.bitcast_convert_type(l3[:, 0, :], jnp.int16)
        b = jax.lax.bitcast_convert_type(l3[:, 1, :], jnp.int16)
        ai = a.astype(jnp.int32) & 0xFFFF
        bi = (b.astype(jnp.int32) & 0xFFFF) << 16
        out_ref[0, :, :] = bi | ai


def _zero(ref, nbins, dtype):
    def body(j, _):
        ref[pl.ds(j * 16, 16)] = jnp.zeros((16,), dtype)
        return 0
    lax.fori_loop(0, nbins // 16, body, 0)


def _scan_hist(ref, nbins, k_rank):
    nv = nbins // 16

    def body(j, carry):
        acc, found, b, sb1 = carry
        jj = nv - 1 - j
        v = ref[pl.ds(jj * 16, 16)]
        svec = lax.rev(jnp.cumsum(lax.rev(v, (0,))), (0,)) + acc
        ge = svec >= k_rank
        trues = jnp.sum(ge.astype(jnp.int32))
        found_now = jnp.logical_and(jnp.logical_not(found), trues > 0)
        sb1_here = jnp.maximum(jnp.max(jnp.where(ge, jnp.int32(-1), svec)), acc)
        b = jnp.where(found_now, jj * 16 + trues - 1, b)
        sb1 = jnp.where(found_now, sb1_here, sb1)
        found = jnp.logical_or(found, trues > 0)
        return acc + jnp.sum(v), found, b, sb1

    _, _, b, sb1 = lax.fori_loop(
        0, nv, body, (jnp.int32(0), False, jnp.int32(0), jnp.int32(0))
    )
    return b, sb1


def _sc_select_body(loss_hbm, out_hbm, buf_v, hist_v, part_v,
                    s3_v, stg_v, res_v, sh_i, sh_f, dma_sem):
    cid = lax.axis_index("c")
    sid = lax.axis_index("s")
    row = cid * 8 + sid // 2
    half = sid % 2
    base = row * (_N // 2) + half * _HALFW
    partner = sid + 1 - 2 * half

    def data_pass(pass_body):
        def start(c):
            return pltpu.async_copy(
                loss_hbm.at[pl.ds(base + c * _CH, _CH)],
                buf_v.at[pl.ds((c % 2) * _CH, _CH)],
                dma_sem,
            )

        carry = jnp.float32(0.0)
        desc = start(0)
        for c in range(_NCH):
            desc.wait()
            if c + 1 < _NCH:
                desc = start(c + 1)
            cbase = (c % 2) * _CH

            def vec(off, carry2):
                x = buf_v[pl.ds(cbase + off, 16)]
                carry2 = pass_body(x & 0xFFFF, carry2)
                return pass_body((x >> 16) & 0xFFFF, carry2)
            carry = plsc.parallel_loop(0, _CH, 16, unroll=_U, carry=carry)(vec)
        return carry

    ones = jnp.ones((16,), jnp.int32)

    def merge_hist(nbins):
        pltpu.sync_copy(hist_v.at[pl.ds(0, nbins)], sh_i.at[sid, pl.ds(0, nbins)])
        plsc.subcore_barrier()
        pltpu.sync_copy(sh_i.at[partner, pl.ds(0, nbins)], part_v.at[pl.ds(0, nbins)])

        def madd(j, _):
            d = pl.ds(j * 16, 16)
            hist_v[d] = hist_v[d] + part_v[d]
            return 0
        lax.fori_loop(0, nbins // 16, madd, 0)
        plsc.subcore_barrier()

    _zero(hist_v, _NB12, jnp.int32)

    def p1(key, carry):
        plsc.addupdate_scatter(hist_v, [key >> 3], ones)
        return carry
    data_pass(p1)
    merge_hist(_NB12)
    b1, c1 = _scan_hist(hist_v, _NB12, jnp.int32(_K))
    k2 = _K - c1

    _zero(hist_v, _NB3, jnp.int32)
    _zero(s3_v, _NB3, jnp.float32)

    def p2(key, sum_hi):
        hi12 = key >> 3
        m_eq = hi12 == b1
        val = plsc.bitcast(key << 16, jnp.float32)
        plsc.addupdate_scatter(hist_v, [key & 7], ones, mask=m_eq)
        plsc.addupdate_scatter(s3_v, [key & 7], val, mask=m_eq)
        return sum_hi + jnp.sum(jnp.where(hi12 > b1, val, 0.0))
    sum_hi = data_pass(p2)

    pltpu.sync_copy(hist_v.at[pl.ds(0, _NB3)], sh_i.at[sid, pl.ds(0, _NB3)])
    stg_v[pl.ds(0, 16)] = s3_v[pl.ds(0, 16)]
    stg_v[pl.ds(16, 16)] = jnp.full((16,), sum_hi, jnp.float32)
    pltpu.sync_copy(stg_v.at[pl.ds(0, 32)], sh_f.at[sid, pl.ds(0, 32)])
    plsc.subcore_barrier()
    pltpu.sync_copy(sh_i.at[partner, pl.ds(0, _NB3)], part_v.at[pl.ds(0, _NB3)])
    pltpu.sync_copy(sh_f.at[partner, pl.ds(0, 32)], stg_v.at[pl.ds(0, 32)])

    hist_v[pl.ds(0, 16)] = hist_v[pl.ds(0, 16)] + part_v[pl.ds(0, 16)]
    s3_v[pl.ds(0, 16)] = s3_v[pl.ds(0, 16)] + stg_v[pl.ds(0, 16)]
    sum_hi = sum_hi + jnp.max(stg_v[pl.ds(16, 16)])

    b2, c2 = _scan_hist(hist_v, _NB3, k2)
    t_bits = (b1 << 3) | b2
    t_val = plsc.bitcast(jnp.full((16,), t_bits << 16, jnp.int32), jnp.float32)

    gidx = lax.iota(jnp.int32, 16)
    sum_in = jnp.sum(jnp.where(gidx > b2, s3_v[pl.ds(0, 16)], 0.0))

    row_sum = sum_hi + sum_in + t_val[0] * (k2 - c2).astype(jnp.float32)

    @pl.when(half == 0)
    def _():
        res_v[...] = jnp.full((16,), row_sum, jnp.float32)
        pltpu.sync_copy(res_v, out_hbm.at[row])


_sc_select = functools.partial(
    pl.kernel,
    out_type=jax.ShapeDtypeStruct((_B, 16), jnp.float32),
    mesh=plsc.VectorSubcoreMesh(core_axis_name="c", subcore_axis_name="s"),
    scratch_types=[
        pltpu.VMEM((2 * _CH,), jnp.int32),
        pltpu.VMEM((_NB12,), jnp.int32),
        pltpu.VMEM((_NB12,), jnp.int32),
        pltpu.VMEM((_NB3,), jnp.float32),
        pltpu.VMEM((256,), jnp.float32),
        pltpu.VMEM((16,), jnp.float32),
        pltpu.VMEM_SHARED((16, _NB12), jnp.int32),
        pltpu.VMEM_SHARED((16, 256), jnp.float32),
        pltpu.SemaphoreType.DMA,
    ],
    compiler_params=pltpu.CompilerParams(needs_layout_passes=False),
)(_sc_select_body)


def kernel(pred_logits, gts, step):
    B, C, H, W = pred_logits.shape

    g3 = gts.reshape(B, _ROWS2D, _LANES)
    x4 = pred_logits.reshape(B, C, _ROWS2D, _LANES)

    losses = pl.pallas_call(
        _count_loss_body,
        grid=(2 * B,),
        in_specs=[
            pl.BlockSpec(
                (1, C, _ROWS2D, _LANES),
                lambda i: (jnp.where(i < _B, 0, i - _B), 0, 0, 0),
            ),
            pl.BlockSpec(
                (1, _ROWS2D, _LANES),
                lambda i: (jnp.where(i < _B, i, i - _B), 0, 0),
            ),
        ],
        out_specs=pl.BlockSpec(
            (1, _ROWS2D // 2, _LANES),
            lambda i: (jnp.where(i < _B, 0, i - _B), 0, 0),
        ),
        out_shape=jax.ShapeDtypeStruct((B, _ROWS2D // 2, _LANES), jnp.int32),
        scratch_shapes=[pltpu.SMEM((1,), jnp.float32)],
    )(x4, g3)

    row_sums = _sc_select(losses.reshape(B * _N // 2))[:, 0]

    ratio_t = jnp.minimum(1.0, jnp.asarray(step, jnp.float32) / _HEM_STEP)
    return jnp.sum(row_sums) / float(B * _K) + 0.0 * ratio_t
